# baseline (speedup 1.0000x reference)
import jax
import jax.numpy as jnp
from jax.experimental import pallas as pl
from jax.experimental.pallas import tpu as pltpu


def _round_up(x, m):
    return ((x + m - 1) // m) * m


def _pad(a, target_shape):
    widths = [(0, t - s) for s, t in zip(a.shape, target_shape)]
    if all(w == (0, 0) for w in widths):
        return a
    return jnp.pad(a, widths)


def _pool_kernel(x_ref, o_ref, acc_ref, *, inv_t):
    t = pl.program_id(1)

    @pl.when(t == 0)
    def _():
        acc_ref[...] = jnp.zeros_like(acc_ref)

    acc_ref[...] += jnp.sum(x_ref[...].astype(jnp.float32), axis=1)

    @pl.when(t == pl.num_programs(1) - 1)
    def _():
        o_ref[...] = acc_ref[...] * inv_t


def _head_kernel(p_ref, w1_ref, b1_ref, w2_ref, b2_ref, wp_ref, bp_ref, o_ref):
    pooled = p_ref[...].astype(w1_ref.dtype)
    h = jnp.dot(pooled, w1_ref[...], preferred_element_type=jnp.float32)
    h = jnp.maximum(h + b1_ref[...], 0.0)
    h = jnp.dot(h.astype(w2_ref.dtype), w2_ref[...],
                preferred_element_type=jnp.float32)
    h = h + b2_ref[...]
    out = jnp.dot(h.astype(wp_ref.dtype), wp_ref[...],
                  preferred_element_type=jnp.float32)
    o_ref[...] = (out + bp_ref[...]).astype(o_ref.dtype)


def _resident(shape, index_map):
    return pl.BlockSpec(shape, index_map, pipeline_mode=pl.Buffered(1))


def kernel(x, w1, b1, w2, b2, w_proj, b_proj):
    import functools

    B, T, D_in = x.shape
    D_out = w_proj.shape[1]
    D_in_p = _round_up(D_in, 128)
    D_out_p = _round_up(D_out, 128)
    itemsize = jnp.dtype(x.dtype).itemsize

    if B >= 16:
        TB = _round_up((B + 1) // 2, 8)
    else:
        TB = _round_up(max(B, 1), 8)
    B_pad = _round_up(B, TB)

    TT = max(8, (9 * 1024 * 1024) // (TB * D_in_p * itemsize) // 8 * 8)
    TT = min(TT, _round_up(T, 8))
    T_pad = _round_up(T, TT)

    x_p = _pad(x, (B_pad, T_pad, D_in_p))
    w1p = _pad(w1, (D_in_p, D_in_p))
    b1p = _pad(b1.reshape(1, -1), (1, D_in_p))
    w2p = _pad(w2, (D_in_p, D_in_p))
    b2p = _pad(b2.reshape(1, -1), (1, D_in_p))
    wpp = _pad(w_proj, (D_in_p, D_out_p))
    bpp = _pad(b_proj.reshape(1, -1), (1, D_out_p))

    grid = (B_pad // TB, T_pad // TT)
    pooled = pl.pallas_call(
        functools.partial(_pool_kernel, inv_t=1.0 / T),
        out_shape=jax.ShapeDtypeStruct((B_pad, D_in_p), jnp.float32),
        grid=grid,
        in_specs=[pl.BlockSpec((TB, TT, D_in_p), lambda b, t: (b, t, 0))],
        out_specs=pl.BlockSpec((TB, D_in_p), lambda b, t: (b, 0)),
        scratch_shapes=[pltpu.VMEM((TB, D_in_p), jnp.float32)],
        compiler_params=pltpu.CompilerParams(
            dimension_semantics=("parallel", "arbitrary"),
            vmem_limit_bytes=48 * 1024 * 1024),
        cost_estimate=pl.CostEstimate(
            flops=int(B_pad * T_pad * D_in_p), transcendentals=0,
            bytes_accessed=int(x_p.size * itemsize + B_pad * D_in_p * 4)),
    )(x_p)

    n_j = 4 if D_out_p % 512 == 0 else (2 if D_out_p % 256 == 0 else 1)
    TJ = D_out_p // n_j
    out = pl.pallas_call(
        _head_kernel,
        out_shape=jax.ShapeDtypeStruct((B_pad, D_out_p), x.dtype),
        grid=(n_j,),
        in_specs=[
            _resident((B_pad, D_in_p), lambda j: (0, 0)),
            _resident((D_in_p, D_in_p), lambda j: (0, 0)),
            _resident((1, D_in_p), lambda j: (0, 0)),
            _resident((D_in_p, D_in_p), lambda j: (0, 0)),
            _resident((1, D_in_p), lambda j: (0, 0)),
            pl.BlockSpec((D_in_p, TJ), lambda j: (0, j)),
            pl.BlockSpec((1, TJ), lambda j: (0, j)),
        ],
        out_specs=pl.BlockSpec((B_pad, TJ), lambda j: (0, j)),
        compiler_params=pltpu.CompilerParams(
            dimension_semantics=("parallel",),
            vmem_limit_bytes=48 * 1024 * 1024),
    )(pooled, w1p, b1p, w2p, b2p, wpp, bpp)

    return out[:B, None, :D_out]


# speedup vs baseline: 1.0286x; 1.0286x over previous
import jax
import jax.numpy as jnp
from jax.experimental import pallas as pl
from jax.experimental.pallas import tpu as pltpu


def _round_up(x, m):
    return ((x + m - 1) // m) * m


def _pad(a, target_shape):
    widths = [(0, t - s) for s, t in zip(a.shape, target_shape)]
    if all(w == (0, 0) for w in widths):
        return a
    return jnp.pad(a, widths)


def _pool_kernel(x_ref, o_ref, acc_ref, *, inv_t):
    t = pl.program_id(1)

    @pl.when(t == 0)
    def _():
        acc_ref[...] = jnp.zeros_like(acc_ref)

    acc_ref[...] += jnp.sum(x_ref[...].astype(jnp.float32), axis=1)

    @pl.when(t == pl.num_programs(1) - 1)
    def _():
        o_ref[...] = acc_ref[...] * inv_t


def _head_kernel(p_ref, w1_ref, b1_ref, w2_ref, b2_ref, wp_ref, bp_ref, o_ref):
    pooled = p_ref[...].astype(w1_ref.dtype)
    h = jnp.dot(pooled, w1_ref[...], preferred_element_type=jnp.float32)
    h = jnp.maximum(h + b1_ref[...], 0.0)
    h = jnp.dot(h.astype(w2_ref.dtype), w2_ref[...],
                preferred_element_type=jnp.float32)
    h = h + b2_ref[...]
    out = jnp.dot(h.astype(wp_ref.dtype), wp_ref[...],
                  preferred_element_type=jnp.float32)
    o_ref[...] = (out + bp_ref[...]).astype(o_ref.dtype)


def _resident(shape, index_map):
    return pl.BlockSpec(shape, index_map, pipeline_mode=pl.Buffered(1))


def kernel(x, w1, b1, w2, b2, w_proj, b_proj):
    import functools

    B, T, D_in = x.shape
    D_out = w_proj.shape[1]
    D_in_p = _round_up(D_in, 128)
    D_out_p = _round_up(D_out, 128)
    itemsize = jnp.dtype(x.dtype).itemsize

    if B >= 16:
        TB = _round_up((B + 1) // 2, 8)
    else:
        TB = _round_up(max(B, 1), 8)
    B_pad = _round_up(B, TB)

    TT = max(8, (8 * 1024 * 1024) // (TB * D_in_p * itemsize) // 8 * 8)
    TT = min(TT, _round_up(T, 8))
    T_pad = _round_up(T, TT)

    x_p = _pad(x, (B_pad, T_pad, D_in_p))
    w1p = _pad(w1, (D_in_p, D_in_p))
    b1p = _pad(b1.reshape(1, -1), (1, D_in_p))
    w2p = _pad(w2, (D_in_p, D_in_p))
    b2p = _pad(b2.reshape(1, -1), (1, D_in_p))
    wpp = _pad(w_proj, (D_in_p, D_out_p))
    bpp = _pad(b_proj.reshape(1, -1), (1, D_out_p))

    grid = (B_pad // TB, T_pad // TT)
    pooled = pl.pallas_call(
        functools.partial(_pool_kernel, inv_t=1.0 / T),
        out_shape=jax.ShapeDtypeStruct((B_pad, D_in_p), jnp.float32),
        grid=grid,
        in_specs=[pl.BlockSpec((TB, TT, D_in_p), lambda b, t: (b, t, 0))],
        out_specs=pl.BlockSpec((TB, D_in_p), lambda b, t: (b, 0)),
        scratch_shapes=[pltpu.VMEM((TB, D_in_p), jnp.float32)],
        compiler_params=pltpu.CompilerParams(
            dimension_semantics=("parallel", "arbitrary"),
            vmem_limit_bytes=48 * 1024 * 1024),
        cost_estimate=pl.CostEstimate(
            flops=int(B_pad * T_pad * D_in_p), transcendentals=0,
            bytes_accessed=int(x_p.size * itemsize + B_pad * D_in_p * 4)),
    )(x_p)

    n_j = 2 if D_out_p % 256 == 0 else 1
    TJ = D_out_p // n_j
    out = pl.pallas_call(
        _head_kernel,
        out_shape=jax.ShapeDtypeStruct((B_pad, D_out_p), x.dtype),
        grid=(n_j,),
        in_specs=[
            _resident((B_pad, D_in_p), lambda j: (0, 0)),
            _resident((D_in_p, D_in_p), lambda j: (0, 0)),
            _resident((1, D_in_p), lambda j: (0, 0)),
            _resident((D_in_p, D_in_p), lambda j: (0, 0)),
            _resident((1, D_in_p), lambda j: (0, 0)),
            pl.BlockSpec((D_in_p, TJ), lambda j: (0, j)),
            pl.BlockSpec((1, TJ), lambda j: (0, j)),
        ],
        out_specs=pl.BlockSpec((B_pad, TJ), lambda j: (0, j)),
        compiler_params=pltpu.CompilerParams(
            dimension_semantics=("parallel",),
            vmem_limit_bytes=48 * 1024 * 1024),
    )(pooled, w1p, b1p, w2p, b2p, wpp, bpp)

    return out[:B, None, :D_out]
